# Initial kernel scaffold; baseline (speedup 1.0000x reference)
import jax
import jax.numpy as jnp
from jax.experimental import pallas as pl


def kernel(x, weight, bias):
    raise NotImplementedError("write your pallas kernel here")


# speedup vs baseline: 1.8727x; 1.8727x over previous
import numpy as np
import jax
import jax.numpy as jnp
from jax.experimental import pallas as pl
from jax.experimental.pallas import tpu as pltpu

_G = 8
_EPS = 0.01
_PREC = jax.lax.Precision.HIGHEST


def _pade5_coeffs():
    m = 5
    c = np.zeros(2 * m + 1)
    c[0] = 1.0
    b = 1.0
    for k in range(1, 2 * m + 1):
        b *= (0.5 - (k - 1)) / k
        c[k] = ((-1) ** k) * b
    A = np.zeros((m, m))
    rhs = np.zeros(m)
    for i in range(m):
        k = m + 1 + i
        for j in range(m):
            A[i, j] = c[k - (j + 1)]
        rhs[i] = -c[k]
    q = np.concatenate([[1.0], np.linalg.solve(A, rhs)])
    p = np.array([sum(q[j] * c[k - j] for j in range(min(k, m) + 1))
                  for k in range(m + 1)])
    return [float(v) for v in p], [float(v) for v in q]


_PP, _QQ = _pade5_coeffs()


def _stats_body(x_ref, s_gram_ref, s_sum_ref):
    xb = x_ref[0]
    gram = jax.lax.dot_general(
        xb, xb, (((1,), (1,)), ((), ())),
        preferred_element_type=jnp.float32, precision=_PREC)
    rsum = jnp.sum(xb, axis=1, keepdims=True)

    @pl.when(pl.program_id(1) == 0)
    def _():
        s_gram_ref[0] = gram
        s_sum_ref[0] = rsum

    @pl.when(pl.program_id(1) != 0)
    def _():
        s_gram_ref[0] = s_gram_ref[0] + gram
        s_sum_ref[0] = s_sum_ref[0] + rsum


def _make_solve_body(nm, m_total):
    inv_m = 1.0 / float(m_total)

    def _solve_body(gram_ref, sum_ref, w_ref, bias_ref, a_ref, b_ref):
        gram = gram_ref[0]
        mu = sum_ref[0] * inv_m
        outer = jax.lax.dot_general(
            mu, mu, (((1,), (1,)), ((), ())),
            preferred_element_type=jnp.float32, precision=_PREC)
        rr = jax.lax.broadcasted_iota(jnp.int32, (nm, nm), 0)
        cc = jax.lax.broadcasted_iota(jnp.int32, (nm, nm), 1)
        eye = jnp.where(rr == cc, 1.0, 0.0)
        cov = gram * inv_m - outer + _EPS * eye
        n2 = jnp.sum(jnp.sum(cov * cov, axis=0, keepdims=True),
                     axis=1, keepdims=True)
        norm = jnp.sqrt(n2)
        p = cov / norm
        papp = eye - p
        p_sqrt = _PP[0] * eye
        q_sqrt = _QQ[0] * eye
        ph = papp
        papp_b = papp.astype(jnp.bfloat16)
        for i in range(5):
            p_sqrt = p_sqrt + _PP[i + 1] * ph
            q_sqrt = q_sqrt + _QQ[i + 1] * ph
            if i < 4:
                ph = jax.lax.dot_general(
                    ph.astype(jnp.bfloat16), papp_b,
                    (((1,), (0,)), ((), ())),
                    preferred_element_type=jnp.float32)
        m_aug = jnp.concatenate([p_sqrt, q_sqrt], axis=1)
        col_iota = jax.lax.broadcasted_iota(jnp.int32, (nm, 1), 0)
        for k in range(nm):
            row = m_aug[k:k + 1, :]
            piv = row[:, k:k + 1]
            row_s = row / piv
            u = m_aug[:, k:k + 1] - jnp.where(col_iota == k, 1.0, 0.0)
            m_aug = m_aug - u * row_s
        x_solve = m_aug[:, nm:]
        sub = x_solve * jax.lax.rsqrt(norm)
        a_mat = w_ref[0] * sub
        b_col = bias_ref[0] - jax.lax.dot_general(
            a_mat, mu, (((1,), (0,)), ((), ())),
            preferred_element_type=jnp.float32, precision=_PREC)
        a_ref[0] = a_mat
        b_ref[0] = b_col

    return _solve_body


def _apply_body(a_ref, b_ref, x_ref, o_ref):
    o_ref[0] = jax.lax.dot_general(
        a_ref[0], x_ref[0], (((1,), (0,)), ((), ())),
        preferred_element_type=jnp.float32, precision=_PREC) + b_ref[0]


def kernel(x, weight, bias):
    n, ch, h, w = x.shape
    nm = ch // _G
    hw = h * w
    m_total = n * hw
    x3 = x.reshape(n, ch, hw)

    gram, sums = pl.pallas_call(
        _stats_body,
        grid=(_G, n),
        in_specs=[pl.BlockSpec((1, nm, hw), lambda g, i: (i, g, 0))],
        out_specs=[pl.BlockSpec((1, nm, nm), lambda g, i: (g, 0, 0)),
                   pl.BlockSpec((1, nm, 1), lambda g, i: (g, 0, 0))],
        out_shape=[jax.ShapeDtypeStruct((_G, nm, nm), jnp.float32),
                   jax.ShapeDtypeStruct((_G, nm, 1), jnp.float32)],
        compiler_params=pltpu.CompilerParams(
            dimension_semantics=("parallel", "arbitrary")),
        name="zca_stats",
    )(x3)

    a_mat, b_col = pl.pallas_call(
        _make_solve_body(nm, m_total),
        grid=(_G,),
        in_specs=[pl.BlockSpec((1, nm, nm), lambda g: (g, 0, 0)),
                  pl.BlockSpec((1, nm, 1), lambda g: (g, 0, 0)),
                  pl.BlockSpec((1, nm, 1), lambda g: (g, 0, 0)),
                  pl.BlockSpec((1, nm, 1), lambda g: (g, 0, 0))],
        out_specs=[pl.BlockSpec((1, nm, nm), lambda g: (g, 0, 0)),
                   pl.BlockSpec((1, nm, 1), lambda g: (g, 0, 0))],
        out_shape=[jax.ShapeDtypeStruct((_G, nm, nm), jnp.float32),
                   jax.ShapeDtypeStruct((_G, nm, 1), jnp.float32)],
        compiler_params=pltpu.CompilerParams(
            dimension_semantics=("arbitrary",)),
        name="zca_solve",
    )(gram, sums, weight, bias)

    out = pl.pallas_call(
        _apply_body,
        grid=(n, _G),
        in_specs=[pl.BlockSpec((1, nm, nm), lambda i, g: (g, 0, 0)),
                  pl.BlockSpec((1, nm, 1), lambda i, g: (g, 0, 0)),
                  pl.BlockSpec((1, nm, hw), lambda i, g: (i, g, 0))],
        out_specs=pl.BlockSpec((1, nm, hw), lambda i, g: (i, g, 0)),
        out_shape=jax.ShapeDtypeStruct((n, ch, hw), jnp.float32),
        compiler_params=pltpu.CompilerParams(
            dimension_semantics=("parallel", "arbitrary")),
        name="zca_apply",
    )(a_mat, b_col, x3)

    return out.reshape(n, ch, h, w)
